# baseline (speedup 1.0000x reference)
import functools

import jax
import jax.numpy as jnp
from jax import lax
from jax.experimental import pallas as pl
from jax.experimental.pallas import tpu as pltpu
from jax.experimental.pallas import tpu_sc as plsc

BATCH = 16384
EMBED_DIM = 64
PANEL = 128

_info = plsc.get_sparse_core_info()
_NC, _NS = _info.num_cores, _info.num_subcores
_NW = _NC * _NS
_B_PER_W = BATCH // _NW

_N_SLOTS = 16
_PANEL_H = 16
_STAGE_W = 128
_N_STAGES = _B_PER_W // _STAGE_W


def _gather_body(utT_hbm, itT_hbm, uidx_hbm, iidx_hbm, euT_hbm, eiT_hbm,
                 uidx_v, iidx_v,
                 pb0, pb1, pb2, pb3, pb4, pb5, pb6, pb7,
                 pb8, pb9, pb10, pb11, pb12, pb13, pb14, pb15, stage,
                 sem0, sem1, sem2, sem3, sem4, sem5, sem6, sem7,
                 sem8, sem9, sem10, sem11, sem12, sem13, sem14, sem15):
    wid = lax.axis_index("s") * _NC + lax.axis_index("c")
    base = pl.multiple_of(wid * _B_PER_W, _B_PER_W)
    pltpu.sync_copy(uidx_hbm.at[pl.ds(base, _B_PER_W)], uidx_v)
    pltpu.sync_copy(iidx_hbm.at[pl.ds(base, _B_PER_W)], iidx_v)

    pbs = (pb0, pb1, pb2, pb3, pb4, pb5, pb6, pb7,
           pb8, pb9, pb10, pb11, pb12, pb13, pb14, pb15)
    sems = (sem0, sem1, sem2, sem3, sem4, sem5, sem6, sem7,
            sem8, sem9, sem10, sem11, sem12, sem13, sem14, sem15)
    iota = lax.broadcasted_iota(jnp.int32, (16,), 0)

    def phase(tab, idxv, outT, h0):
        def fire(r, pb, sem):
            poff = pl.multiple_of(r - (r & (PANEL - 1)), PANEL)
            pltpu.async_copy(tab.at[pl.ds(h0, _PANEL_H), pl.ds(poff, PANEL)],
                             pb, sem)

        def wait(pb, sem):
            pltpu.make_async_copy(tab.at[pl.ds(h0, _PANEL_H), pl.ds(0, PANEL)],
                                  pb, sem).wait()

        v0 = idxv[pl.ds(0, 16)]
        for k in range(_N_SLOTS):
            fire(v0[k], pbs[k], sems[k])

        groups_per_stage = _STAGE_W // 16

        def group_body(g, cur):
            t0 = g * 16
            nxt = idxv[pl.ds(jnp.minimum(t0 + 16, _B_PER_W - 16), 16)]
            for k in range(16):
                s = k % _N_SLOTS
                wait(pbs[s], sems[s])
                r = cur[k]
                c = jnp.broadcast_to(r & (PANEL - 1), (16,))
                cc = jnp.broadcast_to((g % groups_per_stage) * 16 + k, (16,))
                for m in range(_PANEL_H // 16):
                    rows = iota + 16 * m
                    v = plsc.load_gather(pbs[s], [rows, c])
                    plsc.store_scatter(stage, [rows, cc], v)
                tt = t0 + k + _N_SLOTS
                rn = cur[k + _N_SLOTS] if k < 16 - _N_SLOTS else nxt[k - (16 - _N_SLOTS)]

                @pl.when(tt < _B_PER_W)
                def _():
                    fire(rn, pbs[s], sems[s])

            @pl.when(g % groups_per_stage == groups_per_stage - 1)
            def _():
                out = pl.multiple_of(
                    base + (g // groups_per_stage) * _STAGE_W, _STAGE_W)
                pltpu.sync_copy(
                    stage, outT.at[pl.ds(h0, _PANEL_H), pl.ds(out, _STAGE_W)])
            return nxt

        lax.fori_loop(0, _B_PER_W // 16, group_body, v0)

    for h0 in range(0, EMBED_DIM, _PANEL_H):
        phase(utT_hbm, uidx_v, euT_hbm, h0)
    for h0 in range(0, EMBED_DIM, _PANEL_H):
        phase(itT_hbm, iidx_v, eiT_hbm, h0)


_sc_gather = functools.partial(
    pl.kernel,
    mesh=plsc.VectorSubcoreMesh(core_axis_name="c", subcore_axis_name="s"),
    out_type=[
        jax.ShapeDtypeStruct((EMBED_DIM, BATCH), jnp.float32),
        jax.ShapeDtypeStruct((EMBED_DIM, BATCH), jnp.float32),
    ],
    scratch_types=[
        pltpu.VMEM((_B_PER_W,), jnp.int32),
        pltpu.VMEM((_B_PER_W,), jnp.int32),
        pltpu.VMEM((_PANEL_H, PANEL), jnp.float32),
        pltpu.VMEM((_PANEL_H, PANEL), jnp.float32),
        pltpu.VMEM((_PANEL_H, PANEL), jnp.float32),
        pltpu.VMEM((_PANEL_H, PANEL), jnp.float32),
        pltpu.VMEM((_PANEL_H, PANEL), jnp.float32),
        pltpu.VMEM((_PANEL_H, PANEL), jnp.float32),
        pltpu.VMEM((_PANEL_H, PANEL), jnp.float32),
        pltpu.VMEM((_PANEL_H, PANEL), jnp.float32),
        pltpu.VMEM((_PANEL_H, PANEL), jnp.float32),
        pltpu.VMEM((_PANEL_H, PANEL), jnp.float32),
        pltpu.VMEM((_PANEL_H, PANEL), jnp.float32),
        pltpu.VMEM((_PANEL_H, PANEL), jnp.float32),
        pltpu.VMEM((_PANEL_H, PANEL), jnp.float32),
        pltpu.VMEM((_PANEL_H, PANEL), jnp.float32),
        pltpu.VMEM((_PANEL_H, PANEL), jnp.float32),
        pltpu.VMEM((_PANEL_H, PANEL), jnp.float32),
        pltpu.VMEM((_PANEL_H, _STAGE_W), jnp.float32),
        pltpu.SemaphoreType.DMA,
        pltpu.SemaphoreType.DMA,
        pltpu.SemaphoreType.DMA,
        pltpu.SemaphoreType.DMA,
        pltpu.SemaphoreType.DMA,
        pltpu.SemaphoreType.DMA,
        pltpu.SemaphoreType.DMA,
        pltpu.SemaphoreType.DMA,
        pltpu.SemaphoreType.DMA,
        pltpu.SemaphoreType.DMA,
        pltpu.SemaphoreType.DMA,
        pltpu.SemaphoreType.DMA,
        pltpu.SemaphoreType.DMA,
        pltpu.SemaphoreType.DMA,
        pltpu.SemaphoreType.DMA,
        pltpu.SemaphoreType.DMA,
    ],
    compiler_params=pltpu.CompilerParams(needs_layout_passes=False),
)(_gather_body)


def _mlp_body(euT_ref, eiT_ref, w1a_ref, w1b_ref, b1_ref, w2_ref, b2_ref,
              w3_ref, b3_ref, wp_ref, bp_ref, out_ref):
    h = w1a_ref[...] @ euT_ref[...] + w1b_ref[...] @ eiT_ref[...] + b1_ref[...]
    h = jnp.maximum(h, 0.0)
    h = jnp.maximum(w2_ref[...] @ h + b2_ref[...], 0.0)
    h = jnp.maximum(w3_ref[...] @ h + b3_ref[...], 0.0)
    logit = wp_ref[...] @ h + bp_ref[...]
    out_ref[...] = jax.nn.sigmoid(logit)


def _mlp(euT, eiT, w1a, w1b, b1, w2, b2, w3, b3, wp, bp, blk=4096):
    n_blocks = BATCH // blk

    def full(shape):
        zeros = (0,) * len(shape)
        return pl.BlockSpec(shape, lambda i: zeros)

    return pl.pallas_call(
        _mlp_body,
        grid=(n_blocks,),
        in_specs=[
            pl.BlockSpec((EMBED_DIM, blk), lambda i: (0, i)),
            pl.BlockSpec((EMBED_DIM, blk), lambda i: (0, i)),
            full(w1a.shape),
            full(w1b.shape),
            full(b1.shape),
            full(w2.shape),
            full(b2.shape),
            full(w3.shape),
            full(b3.shape),
            full(wp.shape),
            full(bp.shape),
        ],
        out_specs=pl.BlockSpec((1, blk), lambda i: (0, i)),
        out_shape=jax.ShapeDtypeStruct((1, BATCH), jnp.float32),
    )(euT, eiT, w1a, w1b, b1, w2, b2, w3, b3, wp, bp)


def kernel(user, item, embed_user, embed_item, W1, b1, W2, b2, W3, b3, Wp, bp):
    u = user.astype(jnp.int32)
    it = item.astype(jnp.int32)
    euT, eiT = _sc_gather(embed_user.T, embed_item.T, u, it)
    out = _mlp(
        euT, eiT,
        W1[:, :EMBED_DIM], W1[:, EMBED_DIM:], b1.reshape(-1, 1),
        W2, b2.reshape(-1, 1), W3, b3.reshape(-1, 1), Wp, bp.reshape(1, 1),
    )
    return out.reshape(-1)


# speedup vs baseline: 1.3527x; 1.3527x over previous
import functools

import jax
import jax.numpy as jnp
from jax import lax
from jax.experimental import pallas as pl
from jax.experimental.pallas import tpu as pltpu
from jax.experimental.pallas import tpu_sc as plsc

BATCH = 16384
EMBED_DIM = 64
PANEL = 128

_info = plsc.get_sparse_core_info()
_NC, _NS = _info.num_cores, _info.num_subcores
_NW = _NC * _NS
_B_PER_W = BATCH // _NW

_N_SLOTS = 8
_PANEL_H = 32
_STAGE_W = 128
_N_STAGES = _B_PER_W // _STAGE_W


def _gather_body(utT_hbm, itT_hbm, uidx_hbm, iidx_hbm, euT_hbm, eiT_hbm,
                 uidx_v, iidx_v,
                 pb0, pb1, pb2, pb3, pb4, pb5, pb6, pb7, stage,
                 sem0, sem1, sem2, sem3, sem4, sem5, sem6, sem7):
    wid = lax.axis_index("s") * _NC + lax.axis_index("c")
    base = pl.multiple_of(wid * _B_PER_W, _B_PER_W)
    pltpu.sync_copy(uidx_hbm.at[pl.ds(base, _B_PER_W)], uidx_v)
    pltpu.sync_copy(iidx_hbm.at[pl.ds(base, _B_PER_W)], iidx_v)

    pbs = (pb0, pb1, pb2, pb3, pb4, pb5, pb6, pb7)
    sems = (sem0, sem1, sem2, sem3, sem4, sem5, sem6, sem7)
    iota = lax.broadcasted_iota(jnp.int32, (16,), 0)

    def phase(tab, idxv, outT, h0):
        def fire(r, pb, sem):
            poff = pl.multiple_of(r - (r & (PANEL - 1)), PANEL)
            pltpu.async_copy(tab.at[pl.ds(h0, _PANEL_H), pl.ds(poff, PANEL)],
                             pb, sem)

        def wait(pb, sem):
            pltpu.make_async_copy(tab.at[pl.ds(h0, _PANEL_H), pl.ds(0, PANEL)],
                                  pb, sem).wait()

        v0 = idxv[pl.ds(0, 16)]
        for k in range(_N_SLOTS):
            fire(v0[k], pbs[k], sems[k])

        groups_per_stage = _STAGE_W // 16

        def group_body(g, cur):
            t0 = g * 16
            nxt = idxv[pl.ds(jnp.minimum(t0 + 16, _B_PER_W - 16), 16)]
            for k in range(16):
                s = k % _N_SLOTS
                wait(pbs[s], sems[s])
                r = cur[k]
                c = jnp.broadcast_to(r & (PANEL - 1), (16,))
                cc = jnp.broadcast_to((g % groups_per_stage) * 16 + k, (16,))
                for m in range(_PANEL_H // 16):
                    rows = iota + 16 * m
                    v = plsc.load_gather(pbs[s], [rows, c])
                    plsc.store_scatter(stage, [rows, cc], v)
                tt = t0 + k + _N_SLOTS
                rn = cur[k + _N_SLOTS] if k < 16 - _N_SLOTS else nxt[k - (16 - _N_SLOTS)]

                @pl.when(tt < _B_PER_W)
                def _():
                    fire(rn, pbs[s], sems[s])

            @pl.when(g % groups_per_stage == groups_per_stage - 1)
            def _():
                out = pl.multiple_of(
                    base + (g // groups_per_stage) * _STAGE_W, _STAGE_W)
                pltpu.sync_copy(
                    stage, outT.at[pl.ds(h0, _PANEL_H), pl.ds(out, _STAGE_W)])
            return nxt

        lax.fori_loop(0, _B_PER_W // 16, group_body, v0)

    for h0 in range(0, EMBED_DIM, _PANEL_H):
        phase(utT_hbm, uidx_v, euT_hbm, h0)
    for h0 in range(0, EMBED_DIM, _PANEL_H):
        phase(itT_hbm, iidx_v, eiT_hbm, h0)


_sc_gather = functools.partial(
    pl.kernel,
    mesh=plsc.VectorSubcoreMesh(core_axis_name="c", subcore_axis_name="s"),
    out_type=[
        jax.ShapeDtypeStruct((EMBED_DIM, BATCH), jnp.float32),
        jax.ShapeDtypeStruct((EMBED_DIM, BATCH), jnp.float32),
    ],
    scratch_types=[
        pltpu.VMEM((_B_PER_W,), jnp.int32),
        pltpu.VMEM((_B_PER_W,), jnp.int32),
        pltpu.VMEM((_PANEL_H, PANEL), jnp.float32),
        pltpu.VMEM((_PANEL_H, PANEL), jnp.float32),
        pltpu.VMEM((_PANEL_H, PANEL), jnp.float32),
        pltpu.VMEM((_PANEL_H, PANEL), jnp.float32),
        pltpu.VMEM((_PANEL_H, PANEL), jnp.float32),
        pltpu.VMEM((_PANEL_H, PANEL), jnp.float32),
        pltpu.VMEM((_PANEL_H, PANEL), jnp.float32),
        pltpu.VMEM((_PANEL_H, PANEL), jnp.float32),
        pltpu.VMEM((_PANEL_H, _STAGE_W), jnp.float32),
        pltpu.SemaphoreType.DMA,
        pltpu.SemaphoreType.DMA,
        pltpu.SemaphoreType.DMA,
        pltpu.SemaphoreType.DMA,
        pltpu.SemaphoreType.DMA,
        pltpu.SemaphoreType.DMA,
        pltpu.SemaphoreType.DMA,
        pltpu.SemaphoreType.DMA,
    ],
    compiler_params=pltpu.CompilerParams(needs_layout_passes=False),
)(_gather_body)


def _dot(a, b):
    return jax.lax.dot(a.astype(jnp.bfloat16), b.astype(jnp.bfloat16),
                       preferred_element_type=jnp.float32)


def _mlp_body(euT_ref, eiT_ref, w1a_ref, w1b_ref, b1_ref, w2_ref, b2_ref,
              w3_ref, b3_ref, wp_ref, bp_ref, out_ref):
    h = _dot(w1a_ref[...], euT_ref[...]) + _dot(w1b_ref[...], eiT_ref[...]) + b1_ref[...]
    h = jnp.maximum(h, 0.0)
    h = jnp.maximum(_dot(w2_ref[...], h) + b2_ref[...], 0.0)
    h = jnp.maximum(_dot(w3_ref[...], h) + b3_ref[...], 0.0)
    logit = _dot(wp_ref[...], h) + bp_ref[...]
    out_ref[...] = jax.nn.sigmoid(logit)


def _mlp(euT, eiT, w1a, w1b, b1, w2, b2, w3, b3, wp, bp, blk=8192):
    n_blocks = BATCH // blk

    def full(shape):
        zeros = (0,) * len(shape)
        return pl.BlockSpec(shape, lambda i: zeros)

    return pl.pallas_call(
        _mlp_body,
        grid=(n_blocks,),
        in_specs=[
            pl.BlockSpec((EMBED_DIM, blk), lambda i: (0, i)),
            pl.BlockSpec((EMBED_DIM, blk), lambda i: (0, i)),
            full(w1a.shape),
            full(w1b.shape),
            full(b1.shape),
            full(w2.shape),
            full(b2.shape),
            full(w3.shape),
            full(b3.shape),
            full(wp.shape),
            full(bp.shape),
        ],
        out_specs=pl.BlockSpec((1, blk), lambda i: (0, i)),
        out_shape=jax.ShapeDtypeStruct((1, BATCH), jnp.float32),
    )(euT, eiT, w1a, w1b, b1, w2, b2, w3, b3, wp, bp)


def kernel(user, item, embed_user, embed_item, W1, b1, W2, b2, W3, b3, Wp, bp):
    u = user.astype(jnp.int32)
    it = item.astype(jnp.int32)
    euT, eiT = _sc_gather(embed_user.T, embed_item.T, u, it)
    out = _mlp(
        euT, eiT,
        W1[:, :EMBED_DIM], W1[:, EMBED_DIM:], b1.reshape(-1, 1),
        W2, b2.reshape(-1, 1), W3, b3.reshape(-1, 1), Wp, bp.reshape(1, 1),
    )
    return out.reshape(-1)


# speedup vs baseline: 1.4522x; 1.0735x over previous
import functools

import jax
import jax.numpy as jnp
from jax import lax
from jax.experimental import pallas as pl
from jax.experimental.pallas import tpu as pltpu
from jax.experimental.pallas import tpu_sc as plsc

BATCH = 16384
EMBED_DIM = 64
PANEL = 128

_info = plsc.get_sparse_core_info()
_NC, _NS = _info.num_cores, _info.num_subcores
_NW = _NC * _NS
_B_PER_W = BATCH // _NW

_N_SLOTS = 8
_AHEAD = 7
_PANEL_H = 32
_STAGE_W = 128


def _gather_body(utT_hbm, itT_hbm, uidx_hbm, iidx_hbm, euT_hbm, eiT_hbm,
                 uidx_v, iidx_v, pb_all, stage, sems):
    wid = lax.axis_index("s") * _NC + lax.axis_index("c")
    base = pl.multiple_of(wid * _B_PER_W, _B_PER_W)
    pltpu.sync_copy(uidx_hbm.at[pl.ds(base, _B_PER_W)], uidx_v)
    pltpu.sync_copy(iidx_hbm.at[pl.ds(base, _B_PER_W)], iidx_v)

    iota = lax.broadcasted_iota(jnp.int32, (16,), 0)
    gps = _STAGE_W // 16

    def phase(tab, idxv, outT, h0):
        def fire(r, s):
            poff = pl.multiple_of(r - (r & (PANEL - 1)), PANEL)
            pltpu.async_copy(tab.at[pl.ds(h0, _PANEL_H), pl.ds(poff, PANEL)],
                             pb_all.at[s], sems.at[s])

        def wait(s):
            pltpu.make_async_copy(
                tab.at[pl.ds(h0, _PANEL_H), pl.ds(0, PANEL)],
                pb_all.at[0], sems.at[s]).wait()

        v0 = idxv[pl.ds(0, 16)]

        lastf = jnp.int32(-1)
        cntf = jnp.int32(0)
        for k in range(_AHEAD):
            pn = v0[k] >> 7
            new = pn != lastf

            @pl.when(new)
            def _(r=v0[k], cntf=cntf):
                fire(r, cntf & (_N_SLOTS - 1))

            cntf = cntf + new.astype(jnp.int32)
            lastf = pn

        def group_body(g, carry):
            cur, cnte, cntf, laste, lastf = carry
            t0 = g * 16
            nxt = idxv[pl.ds(jnp.minimum(t0 + 16, _B_PER_W - 16), 16)]
            for k in range(16):
                r = cur[k]
                pn = r >> 7
                new_e = pn != laste
                cnte = cnte + new_e.astype(jnp.int32)
                se = (cnte - 1) & (_N_SLOTS - 1)

                @pl.when(new_e)
                def _(se=se):
                    wait(se)

                c = jnp.broadcast_to(r & (PANEL - 1), (16,))
                cc = jnp.broadcast_to((g % gps) * 16 + k, (16,))
                sev = jnp.broadcast_to(se, (16,))
                for m in range(_PANEL_H // 16):
                    rows = iota + 16 * m
                    v = plsc.load_gather(pb_all, [sev, rows, c])
                    plsc.store_scatter(stage, [rows, cc], v)
                laste = pn

                kf = k + _AHEAD
                rf = cur[kf] if kf < 16 else nxt[kf - 16]
                pf = rf >> 7
                tt = t0 + k + _AHEAD
                new_f = (pf != lastf) & (tt < _B_PER_W)

                @pl.when(new_f)
                def _(rf=rf, cntf=cntf):
                    fire(rf, cntf & (_N_SLOTS - 1))

                cntf = cntf + new_f.astype(jnp.int32)
                lastf = pf

            @pl.when(g % gps == gps - 1)
            def _():
                out = pl.multiple_of(base + (g // gps) * _STAGE_W, _STAGE_W)
                pltpu.sync_copy(
                    stage, outT.at[pl.ds(h0, _PANEL_H), pl.ds(out, _STAGE_W)])
            return (nxt, cnte, cntf, laste, lastf)

        init = (v0, jnp.int32(0), cntf, jnp.int32(-1), v0[_AHEAD - 1] >> 7)
        lax.fori_loop(0, _B_PER_W // 16, group_body, init)

    for h0 in range(0, EMBED_DIM, _PANEL_H):
        phase(utT_hbm, uidx_v, euT_hbm, h0)
    for h0 in range(0, EMBED_DIM, _PANEL_H):
        phase(itT_hbm, iidx_v, eiT_hbm, h0)


_sc_gather = functools.partial(
    pl.kernel,
    mesh=plsc.VectorSubcoreMesh(core_axis_name="c", subcore_axis_name="s"),
    out_type=[
        jax.ShapeDtypeStruct((EMBED_DIM, BATCH), jnp.float32),
        jax.ShapeDtypeStruct((EMBED_DIM, BATCH), jnp.float32),
    ],
    scratch_types=[
        pltpu.VMEM((_B_PER_W,), jnp.int32),
        pltpu.VMEM((_B_PER_W,), jnp.int32),
        pltpu.VMEM((_N_SLOTS, _PANEL_H, PANEL), jnp.float32),
        pltpu.VMEM((_PANEL_H, _STAGE_W), jnp.float32),
        pltpu.SemaphoreType.DMA((_N_SLOTS,)),
    ],
    compiler_params=pltpu.CompilerParams(needs_layout_passes=False),
)(_gather_body)


def _dot(a, b):
    return jax.lax.dot(a.astype(jnp.bfloat16), b.astype(jnp.bfloat16),
                       preferred_element_type=jnp.float32)


def _mlp_body(euT_ref, eiT_ref, w1a_ref, w1b_ref, b1_ref, w2_ref, b2_ref,
              w3_ref, b3_ref, wp_ref, bp_ref, out_ref):
    h = _dot(w1a_ref[...], euT_ref[...]) + _dot(w1b_ref[...], eiT_ref[...]) + b1_ref[...]
    h = jnp.maximum(h, 0.0)
    h = jnp.maximum(_dot(w2_ref[...], h) + b2_ref[...], 0.0)
    h = jnp.maximum(_dot(w3_ref[...], h) + b3_ref[...], 0.0)
    logit = _dot(wp_ref[...], h) + bp_ref[...]
    out_ref[...] = jax.nn.sigmoid(logit)


def _mlp(euT, eiT, w1a, w1b, b1, w2, b2, w3, b3, wp, bp, blk=8192):
    n_blocks = BATCH // blk

    def full(shape):
        zeros = (0,) * len(shape)
        return pl.BlockSpec(shape, lambda i: zeros)

    return pl.pallas_call(
        _mlp_body,
        grid=(n_blocks,),
        in_specs=[
            pl.BlockSpec((EMBED_DIM, blk), lambda i: (0, i)),
            pl.BlockSpec((EMBED_DIM, blk), lambda i: (0, i)),
            full(w1a.shape),
            full(w1b.shape),
            full(b1.shape),
            full(w2.shape),
            full(b2.shape),
            full(w3.shape),
            full(b3.shape),
            full(wp.shape),
            full(bp.shape),
        ],
        out_specs=pl.BlockSpec((1, blk), lambda i: (0, i)),
        out_shape=jax.ShapeDtypeStruct((1, BATCH), jnp.float32),
    )(euT, eiT, w1a, w1b, b1, w2, b2, w3, b3, wp, bp)


def kernel(user, item, embed_user, embed_item, W1, b1, W2, b2, W3, b3, Wp, bp):
    u = user.astype(jnp.int32)
    it = item.astype(jnp.int32)
    order = jnp.argsort(u)
    su = u[order]
    si = it[order]
    euT, eiT = _sc_gather(embed_user.T, embed_item.T, su, si)
    out = _mlp(
        euT, eiT,
        W1[:, :EMBED_DIM], W1[:, EMBED_DIM:], b1.reshape(-1, 1),
        W2, b2.reshape(-1, 1), W3, b3.reshape(-1, 1), Wp, bp.reshape(1, 1),
    )
    inv = jnp.zeros((BATCH,), jnp.int32).at[order].set(
        jnp.arange(BATCH, dtype=jnp.int32))
    return out.reshape(-1)[inv]


# speedup vs baseline: 1.4609x; 1.0060x over previous
import functools

import jax
import jax.numpy as jnp
from jax import lax
from jax.experimental import pallas as pl
from jax.experimental.pallas import tpu as pltpu
from jax.experimental.pallas import tpu_sc as plsc

BATCH = 16384
EMBED_DIM = 64
PANEL = 128

_info = plsc.get_sparse_core_info()
_NC, _NS = _info.num_cores, _info.num_subcores
_NW = _NC * _NS
_B_PER_W = BATCH // _NW

_N_SLOTS = 8
_AHEAD = 7
_PANEL_H = 32
_STAGE_W = 128


def _gather_body(utT_hbm, itT_hbm, uidx_hbm, iidx_hbm, euT_hbm, eiT_hbm,
                 uidx_v, iidx_v, pb_all, stage, sems):
    wid = lax.axis_index("s") * _NC + lax.axis_index("c")
    base = pl.multiple_of(wid * _B_PER_W, _B_PER_W)
    pltpu.sync_copy(uidx_hbm.at[pl.ds(base, _B_PER_W)], uidx_v)
    pltpu.sync_copy(iidx_hbm.at[pl.ds(base, _B_PER_W)], iidx_v)

    iota = lax.broadcasted_iota(jnp.int32, (16,), 0)
    gps = _STAGE_W // 16

    def phase(tab, idxv, outT, h0):
        def fire(r, s):
            poff = pl.multiple_of(r - (r & (PANEL - 1)), PANEL)
            pltpu.async_copy(tab.at[pl.ds(h0, _PANEL_H), pl.ds(poff, PANEL)],
                             pb_all.at[s], sems.at[s])

        def wait(s):
            pltpu.make_async_copy(
                tab.at[pl.ds(h0, _PANEL_H), pl.ds(0, PANEL)],
                pb_all.at[0], sems.at[s]).wait()

        v0 = idxv[pl.ds(0, 16)]

        lastf = jnp.int32(-1)
        cntf = jnp.int32(0)
        for k in range(_AHEAD):
            pn = v0[k] >> 7
            new = pn != lastf

            @pl.when(new)
            def _(r=v0[k], cntf=cntf):
                fire(r, cntf & (_N_SLOTS - 1))

            cntf = cntf + new.astype(jnp.int32)
            lastf = pn

        def group_body(g, carry):
            cur, cnte, cntf, laste, lastf = carry
            t0 = g * 16
            nxt = idxv[pl.ds(jnp.minimum(t0 + 16, _B_PER_W - 16), 16)]
            for k in range(16):
                r = cur[k]
                pn = r >> 7
                new_e = pn != laste
                cnte = cnte + new_e.astype(jnp.int32)
                se = (cnte - 1) & (_N_SLOTS - 1)

                @pl.when(new_e)
                def _(se=se):
                    wait(se)

                c = jnp.broadcast_to(r & (PANEL - 1), (16,))
                cc = jnp.broadcast_to((g % gps) * 16 + k, (16,))
                sev = jnp.broadcast_to(se, (16,))
                for m in range(_PANEL_H // 16):
                    rows = iota + 16 * m
                    v = plsc.load_gather(pb_all, [sev, rows, c])
                    plsc.store_scatter(stage, [rows, cc], v)
                laste = pn

                kf = k + _AHEAD
                rf = cur[kf] if kf < 16 else nxt[kf - 16]
                pf = rf >> 7
                tt = t0 + k + _AHEAD
                new_f = (pf != lastf) & (tt < _B_PER_W)

                @pl.when(new_f)
                def _(rf=rf, cntf=cntf):
                    fire(rf, cntf & (_N_SLOTS - 1))

                cntf = cntf + new_f.astype(jnp.int32)
                lastf = pf

            @pl.when(g % gps == gps - 1)
            def _():
                out = pl.multiple_of(base + (g // gps) * _STAGE_W, _STAGE_W)
                pltpu.sync_copy(
                    stage, outT.at[pl.ds(h0, _PANEL_H), pl.ds(out, _STAGE_W)])
            return (nxt, cnte, cntf, laste, lastf)

        init = (v0, jnp.int32(0), cntf, jnp.int32(-1), v0[_AHEAD - 1] >> 7)
        lax.fori_loop(0, _B_PER_W // 16, group_body, init)

    for h0 in range(0, EMBED_DIM, _PANEL_H):
        phase(utT_hbm, uidx_v, euT_hbm, h0)
    for h0 in range(0, EMBED_DIM, _PANEL_H):
        phase(itT_hbm, iidx_v, eiT_hbm, h0)


_sc_gather = functools.partial(
    pl.kernel,
    mesh=plsc.VectorSubcoreMesh(core_axis_name="c", subcore_axis_name="s"),
    out_type=[
        jax.ShapeDtypeStruct((EMBED_DIM, BATCH), jnp.float32),
        jax.ShapeDtypeStruct((EMBED_DIM, BATCH), jnp.float32),
    ],
    scratch_types=[
        pltpu.VMEM((_B_PER_W,), jnp.int32),
        pltpu.VMEM((_B_PER_W,), jnp.int32),
        pltpu.VMEM((_N_SLOTS, _PANEL_H, PANEL), jnp.float32),
        pltpu.VMEM((_PANEL_H, _STAGE_W), jnp.float32),
        pltpu.SemaphoreType.DMA((_N_SLOTS,)),
    ],
    compiler_params=pltpu.CompilerParams(needs_layout_passes=False),
)(_gather_body)


def _dot(a, b):
    return jax.lax.dot(a.astype(jnp.bfloat16), b.astype(jnp.bfloat16),
                       preferred_element_type=jnp.float32)


def _mlp_body(euT_ref, eiT_ref, w1a_ref, w1b_ref, b1_ref, w2_ref, b2_ref,
              w3_ref, b3_ref, wp_ref, bp_ref, out_ref):
    h = _dot(w1a_ref[...], euT_ref[...]) + _dot(w1b_ref[...], eiT_ref[...]) + b1_ref[...]
    h = jnp.maximum(h, 0.0)
    h = jnp.maximum(_dot(w2_ref[...], h) + b2_ref[...], 0.0)
    h = jnp.maximum(_dot(w3_ref[...], h) + b3_ref[...], 0.0)
    logit = _dot(wp_ref[...], h) + bp_ref[...]
    out_ref[...] = jax.nn.sigmoid(logit)


def _mlp(euT, eiT, w1a, w1b, b1, w2, b2, w3, b3, wp, bp, blk=8192):
    n_blocks = BATCH // blk

    def full(shape):
        zeros = (0,) * len(shape)
        return pl.BlockSpec(shape, lambda i: zeros)

    return pl.pallas_call(
        _mlp_body,
        grid=(n_blocks,),
        in_specs=[
            pl.BlockSpec((EMBED_DIM, blk), lambda i: (0, i)),
            pl.BlockSpec((EMBED_DIM, blk), lambda i: (0, i)),
            full(w1a.shape),
            full(w1b.shape),
            full(b1.shape),
            full(w2.shape),
            full(b2.shape),
            full(w3.shape),
            full(b3.shape),
            full(wp.shape),
            full(bp.shape),
        ],
        out_specs=pl.BlockSpec((1, blk), lambda i: (0, i)),
        out_shape=jax.ShapeDtypeStruct((1, BATCH), jnp.float32),
    )(euT, eiT, w1a, w1b, b1, w2, b2, w3, b3, wp, bp)


def kernel(user, item, embed_user, embed_item, W1, b1, W2, b2, W3, b3, Wp, bp):
    u = user.astype(jnp.int32)
    it = item.astype(jnp.int32)
    order = jnp.argsort(u)
    su = u[order]
    si = it[order]
    euT, eiT = _sc_gather(embed_user.T, embed_item.T, su, si)
    out = _mlp(
        euT, eiT,
        W1[:, :EMBED_DIM], W1[:, EMBED_DIM:], b1.reshape(-1, 1),
        W2, b2.reshape(-1, 1), W3, b3.reshape(-1, 1), Wp, bp.reshape(1, 1),
    )
    inv = jnp.argsort(order)
    return out.reshape(-1)[inv]


# speedup vs baseline: 1.5059x; 1.0308x over previous
import functools

import jax
import jax.numpy as jnp
from jax import lax
from jax.experimental import pallas as pl
from jax.experimental.pallas import tpu as pltpu
from jax.experimental.pallas import tpu_sc as plsc

BATCH = 16384
EMBED_DIM = 64
PANEL = 128

_info = plsc.get_sparse_core_info()
_NC, _NS = _info.num_cores, _info.num_subcores
_NW = _NC * _NS
_B_PER_W = BATCH // _NW

_N_SLOTS = 6
_AHEAD = 5
_PANEL_H = 64
_STAGE_W = 128


def _gather_body(utT_hbm, itT_hbm, uidx_hbm, iidx_hbm, euT_hbm, eiT_hbm,
                 uidx_v, iidx_v, pb_all, stage, sems):
    wid = lax.axis_index("s") * _NC + lax.axis_index("c")
    base = pl.multiple_of(wid * _B_PER_W, _B_PER_W)
    pltpu.sync_copy(uidx_hbm.at[pl.ds(base, _B_PER_W)], uidx_v)
    pltpu.sync_copy(iidx_hbm.at[pl.ds(base, _B_PER_W)], iidx_v)

    iota = lax.broadcasted_iota(jnp.int32, (16,), 0)
    gps = _STAGE_W // 16

    def phase(tab, idxv, outT, h0):
        def fire(r, s):
            poff = pl.multiple_of(r - (r & (PANEL - 1)), PANEL)
            pltpu.async_copy(tab.at[pl.ds(h0, _PANEL_H), pl.ds(poff, PANEL)],
                             pb_all.at[s], sems.at[s])

        def wait(s):
            pltpu.make_async_copy(
                tab.at[pl.ds(h0, _PANEL_H), pl.ds(0, PANEL)],
                pb_all.at[0], sems.at[s]).wait()

        v0 = idxv[pl.ds(0, 16)]

        lastf = jnp.int32(-1)
        cntf = jnp.int32(0)
        for k in range(_AHEAD):
            pn = v0[k] >> 7
            new = pn != lastf

            @pl.when(new)
            def _(r=v0[k], cntf=cntf):
                fire(r, cntf % _N_SLOTS)

            cntf = cntf + new.astype(jnp.int32)
            lastf = pn

        def group_body(g, carry):
            cur, cnte, cntf, laste, lastf = carry
            t0 = g * 16
            nxt = idxv[pl.ds(jnp.minimum(t0 + 16, _B_PER_W - 16), 16)]
            for k in range(16):
                r = cur[k]
                pn = r >> 7
                new_e = pn != laste
                cnte = cnte + new_e.astype(jnp.int32)
                se = (cnte - 1) % _N_SLOTS

                @pl.when(new_e)
                def _(se=se):
                    wait(se)

                c = jnp.broadcast_to(r & (PANEL - 1), (16,))
                cc = jnp.broadcast_to((g % gps) * 16 + k, (16,))
                sev = jnp.broadcast_to(se, (16,))
                for m in range(_PANEL_H // 16):
                    rows = iota + 16 * m
                    v = plsc.load_gather(pb_all, [sev, rows, c])
                    plsc.store_scatter(stage, [rows, cc], v)
                laste = pn

                kf = k + _AHEAD
                rf = cur[kf] if kf < 16 else nxt[kf - 16]
                pf = rf >> 7
                tt = t0 + k + _AHEAD
                new_f = (pf != lastf) & (tt < _B_PER_W)

                @pl.when(new_f)
                def _(rf=rf, cntf=cntf):
                    fire(rf, cntf % _N_SLOTS)

                cntf = cntf + new_f.astype(jnp.int32)
                lastf = pf

            @pl.when(g % gps == gps - 1)
            def _():
                out = pl.multiple_of(base + (g // gps) * _STAGE_W, _STAGE_W)
                pltpu.sync_copy(
                    stage, outT.at[pl.ds(h0, _PANEL_H), pl.ds(out, _STAGE_W)])
            return (nxt, cnte, cntf, laste, lastf)

        init = (v0, jnp.int32(0), cntf, jnp.int32(-1), v0[_AHEAD - 1] >> 7)
        lax.fori_loop(0, _B_PER_W // 16, group_body, init)

    for h0 in range(0, EMBED_DIM, _PANEL_H):
        phase(utT_hbm, uidx_v, euT_hbm, h0)
    for h0 in range(0, EMBED_DIM, _PANEL_H):
        phase(itT_hbm, iidx_v, eiT_hbm, h0)


_sc_gather = functools.partial(
    pl.kernel,
    mesh=plsc.VectorSubcoreMesh(core_axis_name="c", subcore_axis_name="s"),
    out_type=[
        jax.ShapeDtypeStruct((EMBED_DIM, BATCH), jnp.float32),
        jax.ShapeDtypeStruct((EMBED_DIM, BATCH), jnp.float32),
    ],
    scratch_types=[
        pltpu.VMEM((_B_PER_W,), jnp.int32),
        pltpu.VMEM((_B_PER_W,), jnp.int32),
        pltpu.VMEM((_N_SLOTS, _PANEL_H, PANEL), jnp.float32),
        pltpu.VMEM((_PANEL_H, _STAGE_W), jnp.float32),
        pltpu.SemaphoreType.DMA((_N_SLOTS,)),
    ],
    compiler_params=pltpu.CompilerParams(needs_layout_passes=False),
)(_gather_body)


def _dot(a, b):
    return jax.lax.dot(a.astype(jnp.bfloat16), b.astype(jnp.bfloat16),
                       preferred_element_type=jnp.float32)


def _mlp_body(euT_ref, eiT_ref, w1a_ref, w1b_ref, b1_ref, w2_ref, b2_ref,
              w3_ref, b3_ref, wp_ref, bp_ref, out_ref):
    h = _dot(w1a_ref[...], euT_ref[...]) + _dot(w1b_ref[...], eiT_ref[...]) + b1_ref[...]
    h = jnp.maximum(h, 0.0)
    h = jnp.maximum(_dot(w2_ref[...], h) + b2_ref[...], 0.0)
    h = jnp.maximum(_dot(w3_ref[...], h) + b3_ref[...], 0.0)
    logit = _dot(wp_ref[...], h) + bp_ref[...]
    out_ref[...] = jax.nn.sigmoid(logit)


def _mlp(euT, eiT, w1a, w1b, b1, w2, b2, w3, b3, wp, bp, blk=8192):
    n_blocks = BATCH // blk

    def full(shape):
        zeros = (0,) * len(shape)
        return pl.BlockSpec(shape, lambda i: zeros)

    return pl.pallas_call(
        _mlp_body,
        grid=(n_blocks,),
        in_specs=[
            pl.BlockSpec((EMBED_DIM, blk), lambda i: (0, i)),
            pl.BlockSpec((EMBED_DIM, blk), lambda i: (0, i)),
            full(w1a.shape),
            full(w1b.shape),
            full(b1.shape),
            full(w2.shape),
            full(b2.shape),
            full(w3.shape),
            full(b3.shape),
            full(wp.shape),
            full(bp.shape),
        ],
        out_specs=pl.BlockSpec((1, blk), lambda i: (0, i)),
        out_shape=jax.ShapeDtypeStruct((1, BATCH), jnp.float32),
    )(euT, eiT, w1a, w1b, b1, w2, b2, w3, b3, wp, bp)


def kernel(user, item, embed_user, embed_item, W1, b1, W2, b2, W3, b3, Wp, bp):
    u = user.astype(jnp.int32)
    it = item.astype(jnp.int32)
    order = jnp.argsort(u)
    su = u[order]
    si = it[order]
    euT, eiT = _sc_gather(embed_user.T, embed_item.T, su, si)
    out = _mlp(
        euT, eiT,
        W1[:, :EMBED_DIM], W1[:, EMBED_DIM:], b1.reshape(-1, 1),
        W2, b2.reshape(-1, 1), W3, b3.reshape(-1, 1), Wp, bp.reshape(1, 1),
    )
    inv = jnp.argsort(order)
    return out.reshape(-1)[inv]


# speedup vs baseline: 1.5060x; 1.0001x over previous
import functools

import jax
import jax.numpy as jnp
from jax import lax
from jax.experimental import pallas as pl
from jax.experimental.pallas import tpu as pltpu
from jax.experimental.pallas import tpu_sc as plsc

BATCH = 16384
EMBED_DIM = 64
PANEL = 128

_NC, _NS = 2, 16
_NW = _NC * _NS
_B_PER_W = BATCH // _NW

_N_SLOTS = 6
_AHEAD = 5
_PANEL_H = 64
_STAGE_W = 128


def _gather_body(utT_hbm, itT_hbm, uidx_hbm, iidx_hbm, euT_hbm, eiT_hbm,
                 uidx_v, iidx_v, pb_all, stage, sems):
    wid = lax.axis_index("s") * _NC + lax.axis_index("c")
    base = pl.multiple_of(wid * _B_PER_W, _B_PER_W)
    pltpu.sync_copy(uidx_hbm.at[pl.ds(base, _B_PER_W)], uidx_v)
    pltpu.sync_copy(iidx_hbm.at[pl.ds(base, _B_PER_W)], iidx_v)

    iota = lax.broadcasted_iota(jnp.int32, (16,), 0)
    gps = _STAGE_W // 16

    def phase(tab, idxv, outT, h0):
        def fire(r, s):
            poff = pl.multiple_of(r - (r & (PANEL - 1)), PANEL)
            pltpu.async_copy(tab.at[pl.ds(h0, _PANEL_H), pl.ds(poff, PANEL)],
                             pb_all.at[s], sems.at[s])

        def wait(s):
            pltpu.make_async_copy(
                tab.at[pl.ds(h0, _PANEL_H), pl.ds(0, PANEL)],
                pb_all.at[0], sems.at[s]).wait()

        v0 = idxv[pl.ds(0, 16)]

        lastf = jnp.int32(-1)
        cntf = jnp.int32(0)
        for k in range(_AHEAD):
            pn = v0[k] >> 7
            new = pn != lastf

            @pl.when(new)
            def _(r=v0[k], cntf=cntf):
                fire(r, cntf % _N_SLOTS)

            cntf = cntf + new.astype(jnp.int32)
            lastf = pn

        def group_body(g, carry):
            cur, cnte, cntf, laste, lastf = carry
            t0 = g * 16
            nxt = idxv[pl.ds(jnp.minimum(t0 + 16, _B_PER_W - 16), 16)]
            for k in range(16):
                r = cur[k]
                pn = r >> 7
                new_e = pn != laste
                cnte = cnte + new_e.astype(jnp.int32)
                se = (cnte - 1) % _N_SLOTS

                @pl.when(new_e)
                def _(se=se):
                    wait(se)

                c = jnp.broadcast_to(r & (PANEL - 1), (16,))
                cc = jnp.broadcast_to((g % gps) * 16 + k, (16,))
                sev = jnp.broadcast_to(se, (16,))
                for m in range(_PANEL_H // 16):
                    rows = iota + 16 * m
                    v = plsc.load_gather(pb_all, [sev, rows, c])
                    plsc.store_scatter(stage, [rows, cc], v)
                laste = pn

                kf = k + _AHEAD
                rf = cur[kf] if kf < 16 else nxt[kf - 16]
                pf = rf >> 7
                tt = t0 + k + _AHEAD
                new_f = (pf != lastf) & (tt < _B_PER_W)

                @pl.when(new_f)
                def _(rf=rf, cntf=cntf):
                    fire(rf, cntf % _N_SLOTS)

                cntf = cntf + new_f.astype(jnp.int32)
                lastf = pf

            @pl.when(g % gps == gps - 1)
            def _():
                out = pl.multiple_of(base + (g // gps) * _STAGE_W, _STAGE_W)
                pltpu.sync_copy(
                    stage, outT.at[pl.ds(h0, _PANEL_H), pl.ds(out, _STAGE_W)])
            return (nxt, cnte, cntf, laste, lastf)

        init = (v0, jnp.int32(0), cntf, jnp.int32(-1), v0[_AHEAD - 1] >> 7)
        lax.fori_loop(0, _B_PER_W // 16, group_body, init)

    for h0 in range(0, EMBED_DIM, _PANEL_H):
        phase(utT_hbm, uidx_v, euT_hbm, h0)
    for h0 in range(0, EMBED_DIM, _PANEL_H):
        phase(itT_hbm, iidx_v, eiT_hbm, h0)


_sc_gather = functools.partial(
    pl.kernel,
    mesh=plsc.VectorSubcoreMesh(core_axis_name="c", subcore_axis_name="s"),
    out_type=[
        jax.ShapeDtypeStruct((EMBED_DIM, BATCH), jnp.float32),
        jax.ShapeDtypeStruct((EMBED_DIM, BATCH), jnp.float32),
    ],
    scratch_types=[
        pltpu.VMEM((_B_PER_W,), jnp.int32),
        pltpu.VMEM((_B_PER_W,), jnp.int32),
        pltpu.VMEM((_N_SLOTS, _PANEL_H, PANEL), jnp.float32),
        pltpu.VMEM((_PANEL_H, _STAGE_W), jnp.float32),
        pltpu.SemaphoreType.DMA((_N_SLOTS,)),
    ],
    compiler_params=pltpu.CompilerParams(needs_layout_passes=False),
)(_gather_body)


def _dot(a, b):
    return jax.lax.dot(a.astype(jnp.bfloat16), b.astype(jnp.bfloat16),
                       preferred_element_type=jnp.float32)


def _mlp_body(euT_ref, eiT_ref, w1a_ref, w1b_ref, b1_ref, w2_ref, b2_ref,
              w3_ref, b3_ref, wp_ref, bp_ref, out_ref):
    h = _dot(w1a_ref[...], euT_ref[...]) + _dot(w1b_ref[...], eiT_ref[...]) + b1_ref[...]
    h = jnp.maximum(h, 0.0)
    h = jnp.maximum(_dot(w2_ref[...], h) + b2_ref[...], 0.0)
    h = jnp.maximum(_dot(w3_ref[...], h) + b3_ref[...], 0.0)
    logit = _dot(wp_ref[...], h) + bp_ref[...]
    out_ref[...] = jax.nn.sigmoid(logit)


def _mlp(euT, eiT, w1a, w1b, b1, w2, b2, w3, b3, wp, bp, blk=8192):
    n_blocks = BATCH // blk

    def full(shape):
        zeros = (0,) * len(shape)
        return pl.BlockSpec(shape, lambda i: zeros)

    return pl.pallas_call(
        _mlp_body,
        grid=(n_blocks,),
        in_specs=[
            pl.BlockSpec((EMBED_DIM, blk), lambda i: (0, i)),
            pl.BlockSpec((EMBED_DIM, blk), lambda i: (0, i)),
            full(w1a.shape),
            full(w1b.shape),
            full(b1.shape),
            full(w2.shape),
            full(b2.shape),
            full(w3.shape),
            full(b3.shape),
            full(wp.shape),
            full(bp.shape),
        ],
        out_specs=pl.BlockSpec((1, blk), lambda i: (0, i)),
        out_shape=jax.ShapeDtypeStruct((1, BATCH), jnp.float32),
    )(euT, eiT, w1a, w1b, b1, w2, b2, w3, b3, wp, bp)


def kernel(user, item, embed_user, embed_item, W1, b1, W2, b2, W3, b3, Wp, bp):
    u = user.astype(jnp.int32)
    it = item.astype(jnp.int32)
    order = jnp.argsort(u)
    su = u[order]
    si = it[order]
    euT, eiT = _sc_gather(embed_user.T, embed_item.T, su, si)
    out = _mlp(
        euT, eiT,
        W1[:, :EMBED_DIM], W1[:, EMBED_DIM:], b1.reshape(-1, 1),
        W2, b2.reshape(-1, 1), W3, b3.reshape(-1, 1), Wp, bp.reshape(1, 1),
    )
    inv = jnp.argsort(order)
    return out.reshape(-1)[inv]
